# Initial kernel scaffold; baseline (speedup 1.0000x reference)
import jax
import jax.numpy as jnp
from jax.experimental import pallas as pl


def kernel(subword_embs, segment_ids):
    raise NotImplementedError("write your pallas kernel here")


# speedup vs baseline: 9.4543x; 9.4543x over previous
import jax
import jax.numpy as jnp
from jax.experimental import pallas as pl

_B, _L, _E, _W = 8, 2048, 768, 1024


def _pool_body(seg_ref, emb_ref, out_ref):
    seg = seg_ref[0, 0, :]
    emb = emb_ref[0]
    wids = jax.lax.broadcasted_iota(jnp.int32, (_W, _L), 0)
    onehot = (seg[None, :] == wids).astype(jnp.float32)
    sums = jnp.dot(onehot, emb, preferred_element_type=jnp.float32)
    counts = jnp.sum(onehot, axis=1, keepdims=True)
    out_ref[0] = jnp.where(counts > 0, sums / jnp.maximum(counts, 1.0), 0.0)


def kernel(subword_embs, segment_ids):
    seg3 = segment_ids.reshape(_B, 1, _L).astype(jnp.int32)
    return pl.pallas_call(
        _pool_body,
        grid=(_B,),
        in_specs=[
            pl.BlockSpec((1, 1, _L), lambda b: (b, 0, 0)),
            pl.BlockSpec((1, _L, _E), lambda b: (b, 0, 0)),
        ],
        out_specs=pl.BlockSpec((1, _W, _E), lambda b: (b, 0, 0)),
        out_shape=jax.ShapeDtypeStruct((_B, _W, _E), jnp.float32),
    )(seg3, subword_embs)
